# baseline (speedup 1.0000x reference)
import jax
import jax.numpy as jnp
from jax.experimental import pallas as pl
from jax.experimental.pallas import tpu as pltpu

_B = 16384
_I = 512
_O = 16
_BLK = 2048


def _fwd_kernel(x_ref, w_ref, o_ref):
    acc = jax.lax.dot_general(
        w_ref[...],
        x_ref[...],
        dimension_numbers=(((1,), (1,)), ((), ())),
        preferred_element_type=jnp.float32,
    )
    o_ref[...] = jax.nn.sigmoid(acc)


@jax.jit
def kernel(x, W):
    out_t = pl.pallas_call(
        _fwd_kernel,
        grid=(_B // _BLK,),
        in_specs=[
            pl.BlockSpec((_BLK, _I), lambda i: (i, 0)),
            pl.BlockSpec((_O, _I), lambda i: (0, 0)),
        ],
        out_specs=pl.BlockSpec((_O, _BLK), lambda i: (0, i)),
        out_shape=jax.ShapeDtypeStruct((_O, _B), jnp.float32),
        compiler_params=pltpu.CompilerParams(
            dimension_semantics=("arbitrary",),
        ),
    )(x, W)
    return out_t.T
